# baseline (speedup 1.0000x reference)
import jax
import jax.numpy as jnp
from jax import lax
from jax.experimental import pallas as pl
from jax.experimental.pallas import tpu as pltpu
from jax.experimental.pallas import tpu_sc as plsc

N = 10000
D = 256
E = 160000
L = 16
NTILES = 32
NPT = 320
N_PAD = NTILES * NPT
CHUNK = 1600
NCHUNK = E // CHUNK
SCAN_STEPS = CHUNK // L
LB = 32
DC = D // LB
SLAB = 128
NSLAB = (N + SLAB - 1) // SLAB
XROWS = 2 * N + 2 * SLAB
CAPM = 7680
CAPL = CAPM // L


def _sc_agg_body(x_hbm, s_hbm, d_hbm, out_hbm,
                 es, ed, pend_src, pend_dst, bsrc, bdst, msrc, mdst,
                 sbuf, accum,
                 sem_e0, sem_e1, sem_s0, sem_s1):
    cid = lax.axis_index("c")
    sid = lax.axis_index("s")
    wid = sid * 2 + cid
    base = wid * NPT

    neg_inf = jnp.full((LB,), -jnp.inf, dtype=jnp.bfloat16)
    lane_base = jnp.arange(L, dtype=jnp.int32) * CAPL

    def rel_body(r, carry0):
        ebase = r * E

        def init_row(i, c2):
            for c in range(DC):
                accum[i, pl.ds(c * LB, LB)] = neg_inf
            return c2
        lax.fori_loop(0, NPT + 1, init_row, 0)

        pltpu.async_copy(s_hbm.at[pl.ds(ebase, CHUNK)],
                         es.at[pl.ds(0, CHUNK)], sem_e0)
        pltpu.async_copy(d_hbm.at[pl.ds(ebase, CHUNK)],
                         ed.at[pl.ds(0, CHUNK)], sem_e0)

        def round_cond(k0):
            return k0 < NCHUNK

        def round_body(k0):
            def chunk_cond(st):
                k, o = st
                return (k < NCHUNK) & (o <= CAPM - CHUNK)

            def chunk_body(st):
                k, o = st
                kb = k & 1
                boff = kb * CHUNK
                off = ebase + k * CHUNK

                def wait_edges(sem):
                    pltpu.make_async_copy(
                        s_hbm.at[pl.ds(off, CHUNK)],
                        es.at[pl.ds(boff, CHUNK)], sem).wait()
                    pltpu.make_async_copy(
                        d_hbm.at[pl.ds(off, CHUNK)],
                        ed.at[pl.ds(boff, CHUNK)], sem).wait()

                @pl.when(kb == 0)
                def _():
                    wait_edges(sem_e0)

                @pl.when(kb == 1)
                def _():
                    wait_edges(sem_e1)

                @pl.when(k + 1 < NCHUNK)
                def _():
                    noff = ebase + (k + 1) * CHUNK
                    nboff = (1 - kb) * CHUNK

                    def issue_edges(sem):
                        pltpu.async_copy(s_hbm.at[pl.ds(noff, CHUNK)],
                                         es.at[pl.ds(nboff, CHUNK)], sem)
                        pltpu.async_copy(d_hbm.at[pl.ds(noff, CHUNK)],
                                         ed.at[pl.ds(nboff, CHUNK)], sem)

                    @pl.when(kb == 0)
                    def _():
                        issue_edges(sem_e1)

                    @pl.when(kb == 1)
                    def _():
                        issue_edges(sem_e0)

                def scan_step(s, cnt_vec):
                    for u in range(2):
                        so = boff + s * 2 * L + u * L
                        sv = es[pl.ds(so, L)]
                        dv = ed[pl.ds(so, L)]
                        dl = dv - base
                        m = (dl >= 0) & (dl < NPT)
                        pos = lane_base + cnt_vec
                        plsc.store_scatter(pend_src, [pos], sv + r * N,
                                           mask=m)
                        plsc.store_scatter(pend_dst, [pos], dl, mask=m)
                        cnt_vec = cnt_vec + m.astype(jnp.int32)
                    return cnt_vec

                cnt_vec = lax.fori_loop(0, SCAN_STEPS // 2, scan_step,
                                        jnp.zeros((L,), jnp.int32))

                for lane in range(L):
                    nl = cnt_vec[lane]

                    def copy_body(i, o_in):
                        v = pend_src[pl.ds(lane * CAPL + i * L, L)]
                        bsrc[pl.ds(o_in + i * L, L)] = v
                        w = pend_dst[pl.ds(lane * CAPL + i * L, L)]
                        bdst[pl.ds(o_in + i * L, L)] = w
                        return o_in
                    lax.fori_loop(0, (nl + L - 1) // L, copy_body, o)
                    o = o + nl
                return k + 1, o

            k1, o = lax.while_loop(chunk_cond, chunk_body,
                                   (k0, jnp.int32(0)))
            bsrc[pl.ds(o, L)] = jnp.full((L,), XROWS, jnp.int32)
            bdst[pl.ds(o, L)] = jnp.full((L,), NPT, jnp.int32)
            nsteps = (o + L - 1) // L

            pltpu.async_copy(
                x_hbm.at[pl.ds(r * N, SLAB)],
                sbuf.at[pl.ds(0, SLAB)], sem_s0)

            def slab_body(s, c3):
                sb = s & 1
                lo = r * N + s * SLAB

                for p in range(2):
                    @pl.when(sb == p)
                    def _(p=p):
                        pltpu.make_async_copy(
                            x_hbm.at[pl.ds(lo, SLAB)],
                            sbuf.at[pl.ds(p * SLAB, SLAB)],
                            (sem_s0, sem_s1)[p]).wait()

                @pl.when(s + 1 < NSLAB)
                def _():
                    nlo = lo + SLAB
                    for p in range(2):
                        @pl.when(sb == p)
                        def _(p=p):
                            pltpu.async_copy(
                                x_hbm.at[pl.ds(nlo, SLAB)],
                                sbuf.at[pl.ds((1 - p) * SLAB, SLAB)],
                                (sem_s0, sem_s1)[1 - p])

                def rstep(t, cnt_vec):
                    sv = bsrc[pl.ds(t * L, L)]
                    dv = bdst[pl.ds(t * L, L)]
                    sl_ = sv - lo
                    m = (sl_ >= 0) & (sl_ < SLAB)
                    pos = lane_base + cnt_vec
                    plsc.store_scatter(pend_src, [pos], sl_, mask=m)
                    plsc.store_scatter(pend_dst, [pos], dv, mask=m)
                    return cnt_vec + m.astype(jnp.int32)

                cnt2 = lax.fori_loop(0, nsteps, rstep,
                                     jnp.zeros((L,), jnp.int32))

                o2 = jnp.int32(0)
                for lane in range(L):
                    nl = cnt2[lane]

                    def copy2(i, o_in):
                        v = pend_src[pl.ds(lane * CAPL + i * L, L)]
                        msrc[pl.ds(o_in + i * L, L)] = v
                        w = pend_dst[pl.ds(lane * CAPL + i * L, L)]
                        mdst[pl.ds(o_in + i * L, L)] = w
                        return o_in
                    lax.fori_loop(0, (nl + L - 1) // L, copy2, o2)
                    o2 = o2 + nl
                msrc[pl.ds(o2, L)] = jnp.zeros((L,), jnp.int32)
                mdst[pl.ds(o2, L)] = jnp.full((L,), NPT, jnp.int32)
                np2 = (o2 + L - 1) // L

                rbase = sb * SLAB

                def pstep(g, c4):
                    svec = msrc[pl.ds(g * L, L)] + rbase
                    dvec = mdst[pl.ds(g * L, L)]
                    for j in range(L):
                        sj = svec[j]
                        d = dvec[j]
                        sls = [pl.ds(c * LB, LB) for c in range(DC)]
                        rv = [sbuf[sj, sl] for sl in sls]
                        av = [accum[d, sl] for sl in sls]
                        for c in range(DC):
                            accum[d, sls[c]] = jnp.maximum(av[c], rv[c])
                    return c4
                lax.fori_loop(0, np2, pstep, 0)
                return c3
            lax.fori_loop(0, NSLAB, slab_body, 0)
            return k1

        lax.while_loop(round_cond, round_body, jnp.int32(0))

        def fin_row(i, c2):
            for c in range(DC):
                sl = pl.ds(c * LB, LB)
                v = accum[i, sl]
                accum[i, sl] = jnp.where(v == neg_inf,
                                         jnp.bfloat16(0.0), v)
            return c2
        lax.fori_loop(0, NPT, fin_row, 0)
        pltpu.sync_copy(accum.at[pl.ds(0, NPT)],
                        out_hbm.at[r, pl.ds(base, NPT)])
        return carry0
    lax.fori_loop(0, 2, rel_body, 0)


def _sc_aggregate(x2, src2, dst2):
    mesh = plsc.VectorSubcoreMesh(core_axis_name="c", subcore_axis_name="s")
    return pl.kernel(
        _sc_agg_body,
        out_type=jax.ShapeDtypeStruct((2, N_PAD, D), jnp.bfloat16),
        mesh=mesh,
        scratch_types=[
            pltpu.VMEM((2 * CHUNK,), jnp.int32),
            pltpu.VMEM((2 * CHUNK,), jnp.int32),
            pltpu.VMEM((CAPM,), jnp.int32),
            pltpu.VMEM((CAPM,), jnp.int32),
            pltpu.VMEM((CAPM + L,), jnp.int32),
            pltpu.VMEM((CAPM + L,), jnp.int32),
            pltpu.VMEM((CAPM + L,), jnp.int32),
            pltpu.VMEM((CAPM + L,), jnp.int32),
            pltpu.VMEM((2 * SLAB, D), jnp.bfloat16),
            pltpu.VMEM((NPT + 1, D), jnp.bfloat16),
            pltpu.SemaphoreType.DMA,
            pltpu.SemaphoreType.DMA,
            pltpu.SemaphoreType.DMA,
            pltpu.SemaphoreType.DMA,
        ],
        compiler_params=pltpu.CompilerParams(needs_layout_passes=False,
                                             use_tc_tiling_on_sc=False),
    )(x2, src2, dst2)


def _mm_body(xa_ref, xp_ref, aw_ref, ac_ref, wra_ref, bra_ref, wrp_ref,
             brp_ref, ww_ref, wc_ref, oa_ref, op_ref):
    dn = (((1,), (1,)), ((), ()))
    oa_ref[...] = lax.dot_general(
        xa_ref[...], wra_ref[...], dn, preferred_element_type=jnp.float32
    ) + bra_ref[...]
    op_ref[...] = (
        lax.dot_general(xp_ref[...], wrp_ref[...], dn,
                        preferred_element_type=jnp.float32)
        + brp_ref[...]
        + lax.dot_general(aw_ref[...].astype(jnp.float32), ww_ref[...], dn,
                          preferred_element_type=jnp.float32)
        + lax.dot_general(ac_ref[...].astype(jnp.float32), wc_ref[...], dn,
                          preferred_element_type=jnp.float32)
    )


def _tc_matmuls(xa, xp, agg_w, agg_c, wra, bra, wrp, brp, ww, wc):
    bm = 1000
    grid = (N // bm,)
    row_spec = pl.BlockSpec((bm, D), lambda i: (i, 0))
    w_spec = pl.BlockSpec((D, D), lambda i: (0, 0))
    b_spec = pl.BlockSpec((1, D), lambda i: (0, 0))
    return pl.pallas_call(
        _mm_body,
        grid=grid,
        in_specs=[row_spec, row_spec, row_spec, row_spec,
                  w_spec, b_spec, w_spec, b_spec, w_spec, w_spec],
        out_specs=[row_spec, row_spec],
        out_shape=[jax.ShapeDtypeStruct((N, D), jnp.float32)] * 2,
    )(xa, xp, agg_w, agg_c, wra, bra.reshape(1, D), wrp, brp.reshape(1, D),
      ww, wc)


@jax.jit
def kernel(x_author, x_paper, edge_index_writes, edge_index_cites,
           W_writes, W_cites, W_root_author, b_root_author,
           W_root_paper, b_root_paper):
    x2 = jnp.concatenate([x_author, x_paper], axis=0).astype(jnp.bfloat16)
    x2 = jnp.pad(x2, ((0, XROWS - 2 * N), (0, 0)))
    src2 = jnp.concatenate([edge_index_writes[0], edge_index_cites[0]])
    dst2 = jnp.concatenate([edge_index_writes[1], edge_index_cites[1]])
    agg = _sc_aggregate(x2, src2, dst2)
    out_author, out_paper = _tc_matmuls(
        x_author, x_paper, agg[0, :N], agg[1, :N],
        W_root_author, b_root_author, W_root_paper, b_root_paper,
        W_writes, W_cites,
    )
    return (out_author, out_paper)
